# baseline (device time: 35859 ns/iter reference)
import jax
import jax.numpy as jnp
from jax import lax
from jax.experimental import pallas as pl
from jax.experimental.pallas import tpu as pltpu

N_DEV = 16
N_STAGES = 4
NEG_INF = -1e9


def kernel(x, Wq, K_ext, V_ext, Wo):
    B, Sq, Dm = x.shape
    _, skv, Hq, Dh = K_ext.shape
    BH = B * Hq
    Dq = Wq.shape[1]

    xb = x.reshape(B * Sq, Dm).astype(jnp.bfloat16)
    Wqb = Wq.astype(jnp.bfloat16)
    Kb = K_ext.transpose(0, 2, 1, 3).reshape(BH, skv, Dh).astype(jnp.bfloat16)
    Vb = V_ext.transpose(0, 2, 1, 3).reshape(BH, skv, Dh).astype(jnp.bfloat16)
    Wob = Wo.astype(jnp.bfloat16)

    def body(x_ref, wq_ref, k_ref, v_ref, wo_ref, out_ref,
             q_ref, ctx_ref, accf_ref, ml_ref, acc_send_ref,
             acc_recv_ref, ml_recv_ref,
             acc_send_sems, acc_recv_sems, ml_send_sems, ml_recv_sems):
        my = lax.axis_index("i")

        barrier = pltpu.get_barrier_semaphore()
        for s in range(N_STAGES):
            pl.semaphore_signal(barrier, inc=1, device_id=(my ^ (1 << s),),
                                device_id_type=pl.DeviceIdType.MESH)
        pl.semaphore_wait(barrier, N_STAGES)

        q = jnp.dot(x_ref[...], wq_ref[...], preferred_element_type=jnp.float32)
        q_ref[...] = (q * 0.125).astype(jnp.bfloat16)

        row = lax.broadcasted_iota(jnp.int32, (Sq, skv), 0)
        col = lax.broadcasted_iota(jnp.int32, (Sq, skv), 1)
        qb = row // 64
        kb = col // 64 + 2 * my
        mask = (qb == kb) | (kb == 0) | ((qb + kb) % 3 == 0)

        for bh in range(BH):
            b, h = bh // Hq, bh % Hq
            qbh = q_ref[pl.ds(b * Sq, Sq), pl.ds(h * Dh, Dh)]
            sc = lax.dot_general(qbh, k_ref[bh], (((1,), (1,)), ((), ())),
                                 preferred_element_type=jnp.float32)
            m = jnp.max(jnp.where(mask, sc, NEG_INF), axis=1)
            w = jnp.where(mask, jnp.exp(sc - m[:, None]), 0.0)
            l = jnp.sum(w, axis=1)
            accf_ref[bh] = lax.dot_general(
                w.astype(jnp.bfloat16), v_ref[bh], (((1,), (0,)), ((), ())),
                preferred_element_type=jnp.float32)
            ml_ref[0, bh] = m
            ml_ref[1, bh] = l
        acc_send_ref[...] = accf_ref[...].astype(jnp.bfloat16)

        for s in range(N_STAGES):
            partner = my ^ (1 << s)
            rdma_acc = pltpu.make_async_remote_copy(
                src_ref=acc_send_ref, dst_ref=acc_recv_ref.at[s],
                send_sem=acc_send_sems.at[s], recv_sem=acc_recv_sems.at[s],
                device_id=(partner,), device_id_type=pl.DeviceIdType.MESH)
            rdma_ml = pltpu.make_async_remote_copy(
                src_ref=ml_ref, dst_ref=ml_recv_ref.at[s],
                send_sem=ml_send_sems.at[s], recv_sem=ml_recv_sems.at[s],
                device_id=(partner,), device_id_type=pl.DeviceIdType.MESH)
            rdma_acc.start()
            rdma_ml.start()
            rdma_acc.wait()
            rdma_ml.wait()

            m1 = ml_ref[0]
            l1 = ml_ref[1]
            m2 = ml_recv_ref[s, 0]
            l2 = ml_recv_ref[s, 1]
            mn = jnp.maximum(m1, m2)
            a1 = jnp.exp(m1 - mn)
            a2 = jnp.exp(m2 - mn)
            ml_ref[0] = mn
            ml_ref[1] = l1 * a1 + l2 * a2
            accf_ref[...] = (accf_ref[...] * a1[:, :, None]
                             + acc_recv_ref[s].astype(jnp.float32) * a2[:, :, None])
            if s < N_STAGES - 1:
                acc_send_ref[...] = accf_ref[...].astype(jnp.bfloat16)

        linv = 1.0 / ml_ref[1]
        for bh in range(BH):
            b, h = bh // Hq, bh % Hq
            ctx_ref[pl.ds(b * Sq, Sq), pl.ds(h * Dh, Dh)] = (
                accf_ref[bh] * linv[bh][:, None]).astype(jnp.bfloat16)
        out = jnp.dot(ctx_ref[...], wo_ref[...],
                      preferred_element_type=jnp.float32)
        out_ref[...] = out.reshape(B, Sq, Dm)

        def _exit_barrier(bar2):
            for s in range(N_STAGES):
                pl.semaphore_signal(bar2, inc=1, device_id=(my ^ (1 << s),),
                                    device_id_type=pl.DeviceIdType.MESH)
            pl.semaphore_wait(bar2, N_STAGES)
        pl.run_scoped(_exit_barrier, pltpu.SemaphoreType.REGULAR)

    return pl.pallas_call(
        body,
        out_shape=jax.ShapeDtypeStruct((B, Sq, Dm), jnp.float32),
        in_specs=[pl.BlockSpec(memory_space=pltpu.VMEM)] * 5,
        out_specs=pl.BlockSpec(memory_space=pltpu.VMEM),
        scratch_shapes=[
            pltpu.VMEM((B * Sq, Dq), jnp.bfloat16),
            pltpu.VMEM((B * Sq, Dq), jnp.bfloat16),
            pltpu.VMEM((BH, Sq, Dh), jnp.float32),
            pltpu.VMEM((2, BH, Sq), jnp.float32),
            pltpu.VMEM((BH, Sq, Dh), jnp.bfloat16),
            pltpu.VMEM((N_STAGES, BH, Sq, Dh), jnp.bfloat16),
            pltpu.VMEM((N_STAGES, 2, BH, Sq), jnp.float32),
            pltpu.SemaphoreType.DMA((N_STAGES,)),
            pltpu.SemaphoreType.DMA((N_STAGES,)),
            pltpu.SemaphoreType.DMA((N_STAGES,)),
            pltpu.SemaphoreType.DMA((N_STAGES,)),
        ],
        compiler_params=pltpu.CompilerParams(collective_id=0),
    )(xb, Wqb, Kb, Vb, Wob)
